# baseline (speedup 1.0000x reference)
import jax
import jax.numpy as jnp
from jax import lax
from jax.experimental import pallas as pl
from jax.experimental.pallas import tpu as pltpu
from jax.experimental.pallas import tpu_sc as plsc

B, H, W, C = 8, 112, 112, 96
HO, WO = 224, 224
CW = 16
NCG = C // CW
NSUB = 16
HPT = H // NSUB
YPT = HO // NSUB
POS = HPT * W
NROW = POS // 8
CHUNK = HO * WO * CW
SLICE = CHUNK // NSUB
NZ = SLICE // 16
ROWW = WO * CW


NCHUNK = B // 2 * NCG


def _body(x_hbm, idx_hbm, zo_hbm, out_hbm,
          raw_val, raw_idx, svals, soffs, zeros, zo_v, chunk,
          sem_in, sem_sc, sem_fl):
    c = lax.axis_index("c")
    s = lax.axis_index("s")
    lane = lax.iota(jnp.int32, 16)
    h0 = s * HPT
    y0 = s * YPT

    @pl.loop(0, NZ // 16)
    def _zero_init(i):
        zeros[pl.ds(i * 16, 16)] = jnp.zeros((16,), jnp.float32)

    pltpu.sync_copy(zo_hbm, zo_v)
    zo = zo_v[...]

    def fire_loads(k, p):
        b = c * (B // 2) + k // NCG
        cg = k - (k // NCG) * NCG
        pltpu.async_copy(
            x_hbm.at[cg, b, pl.ds(h0, HPT), :], raw_val.at[p], sem_in)
        pltpu.async_copy(
            idx_hbm.at[cg, b, pl.ds(h0, HPT), :], raw_idx.at[p], sem_in)

    def wait_loads(p):
        pltpu.make_async_copy(
            x_hbm.at[0, 0, pl.ds(0, HPT), :], raw_val.at[p], sem_in).wait()
        pltpu.make_async_copy(
            idx_hbm.at[0, 0, pl.ds(0, HPT), :], raw_idx.at[p], sem_in).wait()

    fire_loads(0, 0)
    for z in range(16):
        pltpu.sync_copy(zeros, chunk.at[pl.ds(s * SLICE + z * NZ, NZ)])
    plsc.subcore_barrier()

    @pl.loop(0, NCHUNK)
    def _chunk_loop(k):
        b = c * (B // 2) + k // NCG
        cg = k - (k // NCG) * NCG
        p = k - (k // 2) * 2

        wait_loads(p)

        @pl.when(k < NCHUNK - 1)
        def _prefetch():
            fire_loads(k + 1, 1 - p)

        @pl.loop(0, HPT)
        def _h_loop(hh):
            @pl.loop(0, W // 8)
            def _w_loop(wb):
                row = hh * (W // 8) + wb
                for pp in range(8):
                    vi = raw_idx[p, hh, pl.ds((wb * 8 + pp) * CW, 16)]
                    vv = raw_val[p, hh, pl.ds((wb * 8 + pp) * CW, 16)]
                    q = lax.div(lax.add(vi, zo), jnp.full((16,), C, jnp.int32))
                    off = lax.add(lax.mul(q, jnp.full((16,), CW, jnp.int32)),
                                  lane)
                    soffs[row, pl.ds(pp * 16, 16)] = off
                    svals[row, pl.ds(pp * 16, 16)] = vv
                pltpu.async_copy(svals.at[row], chunk.at[soffs.at[row]],
                                 sem_sc, add=True)

        @pl.loop(0, NROW)
        def _drain(r):
            pltpu.make_async_copy(svals.at[0], chunk.at[soffs.at[0]],
                                  sem_sc).wait()

        plsc.subcore_barrier()

        obase = ((cg * B + b) * HO + y0) * ROWW

        @pl.loop(0, YPT)
        def _flush_fire(yy):
            pltpu.async_copy(chunk.at[pl.ds((y0 + yy) * ROWW, ROWW)],
                             out_hbm.at[pl.ds(obase + yy * ROWW, ROWW)],
                             sem_fl)

        @pl.loop(0, YPT)
        def _flush_drain(yy):
            pltpu.make_async_copy(chunk.at[pl.ds(y0 * ROWW, ROWW)],
                                  out_hbm.at[pl.ds(obase, ROWW)],
                                  sem_fl).wait()

        @pl.loop(0, 16)
        def _zero_fire(z):
            pltpu.async_copy(zeros, chunk.at[pl.ds(s * SLICE + z * NZ, NZ)],
                             sem_fl)

        @pl.loop(0, 16)
        def _zero_drain(z):
            pltpu.make_async_copy(zeros, chunk.at[pl.ds(s * SLICE, NZ)],
                                  sem_fl).wait()

        plsc.subcore_barrier()


@jax.jit
def _unpool(x, idx, zo16):
    x = (x.reshape(B, H, W, NCG, CW).transpose(3, 0, 1, 2, 4)
         .reshape(NCG, B, H, W * CW))
    idx = (idx.reshape(B, H, W, NCG, CW).transpose(3, 0, 1, 2, 4)
           .reshape(NCG, B, H, W * CW))
    fn = pl.kernel(
        _body,
        out_type=jax.ShapeDtypeStruct((NCG * B * HO * ROWW,), jnp.float32),
        mesh=plsc.VectorSubcoreMesh(core_axis_name="c", subcore_axis_name="s"),
        compiler_params=pltpu.CompilerParams(use_tc_tiling_on_sc=False),
        scratch_types=[
            pltpu.VMEM((2, HPT, W * CW), jnp.float32),
            pltpu.VMEM((2, HPT, W * CW), jnp.int32),
            pltpu.VMEM((NROW, 128), jnp.float32),
            pltpu.VMEM((NROW, 128), jnp.int32),
            pltpu.VMEM((NZ,), jnp.float32),
            pltpu.VMEM((16,), jnp.int32),
            pltpu.VMEM_SHARED((CHUNK,), jnp.float32),
            pltpu.SemaphoreType.DMA,
            pltpu.SemaphoreType.DMA,
            pltpu.SemaphoreType.DMA,
        ],
    )
    out_t = fn(x, idx, zo16)
    one = (zo16[0] - zo16[1] + 1).astype(jnp.float32)
    return (out_t.reshape(NCG, B, HO, WO, CW)
            .transpose(1, 2, 3, 0, 4)
            .reshape(B, HO, WO, C)) * one


def kernel(inputs, pooling_indices, output_shape):
    shape_arr = jnp.asarray(output_shape).astype(jnp.int32)
    zo = jnp.sum(shape_arr) - jnp.int32(B + HO + WO + C)
    zo16 = jnp.broadcast_to(zo, (16,)).astype(jnp.int32)
    return _unpool(inputs, pooling_indices.astype(jnp.int32), zo16)


# speedup vs baseline: 1.0204x; 1.0204x over previous
import jax
import jax.numpy as jnp
from jax import lax
from jax.experimental import pallas as pl
from jax.experimental.pallas import tpu as pltpu
from jax.experimental.pallas import tpu_sc as plsc

B, H, W, C = 8, 112, 112, 96
HO, WO = 224, 224
CW = 16
NCG = C // CW
NSUB = 16
HPT = H // NSUB
YPT = HO // NSUB
POS = HPT * W
NROW = POS // 8
CHUNK = HO * WO * CW
SLICE = CHUNK // NSUB
NZ = SLICE // 16
ROWW = WO * CW


NCHUNK = B // 2 * NCG


def _body(x_hbm, idx_hbm, zo_hbm, out_hbm,
          raw_val, raw_idx, svals, soffs, zeros, zo_v, chunk,
          sem_in, sem_sc, sem_fl):
    c = lax.axis_index("c")
    s = lax.axis_index("s")
    lane = lax.iota(jnp.int32, 16)
    h0 = s * HPT
    y0 = s * YPT

    @pl.loop(0, NZ // 16)
    def _zero_init(i):
        zeros[pl.ds(i * 16, 16)] = jnp.zeros((16,), jnp.float32)

    pltpu.sync_copy(zo_hbm, zo_v)
    zo = zo_v[...]

    def fire_loads(k, p):
        b = c * (B // 2) + k // NCG
        cg = k - (k // NCG) * NCG
        pltpu.async_copy(
            x_hbm.at[cg, b, pl.ds(h0, HPT), :], raw_val.at[p], sem_in)
        pltpu.async_copy(
            idx_hbm.at[cg, b, pl.ds(h0, HPT), :], raw_idx.at[p], sem_in)

    def wait_loads(p):
        pltpu.make_async_copy(
            x_hbm.at[0, 0, pl.ds(0, HPT), :], raw_val.at[p], sem_in).wait()
        pltpu.make_async_copy(
            idx_hbm.at[0, 0, pl.ds(0, HPT), :], raw_idx.at[p], sem_in).wait()

    fire_loads(0, 0)
    for z in range(16):
        pltpu.sync_copy(zeros, chunk.at[pl.ds(s * SLICE + z * NZ, NZ)])
    plsc.subcore_barrier()

    @pl.loop(0, NCHUNK)
    def _chunk_loop(k):
        b = c * (B // 2) + k // NCG
        cg = k - (k // NCG) * NCG
        p = k - (k // 2) * 2

        wait_loads(p)

        @pl.when(k < NCHUNK - 1)
        def _prefetch():
            fire_loads(k + 1, 1 - p)

        @pl.loop(0, HPT)
        def _h_loop(hh):
            @pl.loop(0, W // 8)
            def _w_loop(wb):
                row = hh * (W // 8) + wb
                for pp in range(8):
                    vi = raw_idx[p, hh, pl.ds((wb * 8 + pp) * CW, 16)]
                    vv = raw_val[p, hh, pl.ds((wb * 8 + pp) * CW, 16)]
                    q = lax.div(lax.add(vi, zo), jnp.full((16,), C, jnp.int32))
                    off = lax.add(lax.mul(q, jnp.full((16,), CW, jnp.int32)),
                                  lane)
                    soffs[row, pl.ds(pp * 16, 16)] = off
                    svals[row, pl.ds(pp * 16, 16)] = vv
                pltpu.async_copy(svals.at[row], chunk.at[soffs.at[row]],
                                 sem_sc, add=True)

        @pl.loop(0, NROW)
        def _drain(r):
            pltpu.make_async_copy(svals.at[0], chunk.at[soffs.at[0]],
                                  sem_sc).wait()

        plsc.subcore_barrier()

        @pl.loop(0, YPT)
        def _flush_fire(yy):
            pltpu.async_copy(chunk.at[pl.ds((y0 + yy) * ROWW, ROWW)],
                             out_hbm.at[cg, b, y0 + yy, :], sem_fl)

        @pl.loop(0, YPT)
        def _flush_drain(yy):
            pltpu.make_async_copy(chunk.at[pl.ds(y0 * ROWW, ROWW)],
                                  out_hbm.at[cg, b, y0, :], sem_fl).wait()

        @pl.loop(0, 16)
        def _zero_fire(z):
            pltpu.async_copy(zeros, chunk.at[pl.ds(s * SLICE + z * NZ, NZ)],
                             sem_fl)

        @pl.loop(0, 16)
        def _zero_drain(z):
            pltpu.make_async_copy(zeros, chunk.at[pl.ds(s * SLICE, NZ)],
                                  sem_fl).wait()

        plsc.subcore_barrier()


@jax.jit
def _unpool(x, idx, zo16):
    x = (x.reshape(B, H, W, NCG, CW).transpose(3, 0, 1, 2, 4)
         .reshape(NCG, B, H, W * CW))
    idx = (idx.reshape(B, H, W, NCG, CW).transpose(3, 0, 1, 2, 4)
           .reshape(NCG, B, H, W * CW))
    fn = pl.kernel(
        _body,
        out_type=jax.ShapeDtypeStruct((NCG, B, HO, ROWW), jnp.float32),
        mesh=plsc.VectorSubcoreMesh(core_axis_name="c", subcore_axis_name="s"),
        compiler_params=pltpu.CompilerParams(use_tc_tiling_on_sc=False),
        scratch_types=[
            pltpu.VMEM((2, HPT, W * CW), jnp.float32),
            pltpu.VMEM((2, HPT, W * CW), jnp.int32),
            pltpu.VMEM((NROW, 128), jnp.float32),
            pltpu.VMEM((NROW, 128), jnp.int32),
            pltpu.VMEM((NZ,), jnp.float32),
            pltpu.VMEM((16,), jnp.int32),
            pltpu.VMEM_SHARED((CHUNK,), jnp.float32),
            pltpu.SemaphoreType.DMA,
            pltpu.SemaphoreType.DMA,
            pltpu.SemaphoreType.DMA,
        ],
    )
    out_t = fn(x, idx, zo16)
    return (out_t.reshape(NCG, B, HO, WO, CW)
            .transpose(1, 2, 3, 0, 4)
            .reshape(B, HO, WO, C))


def kernel(inputs, pooling_indices, output_shape):
    shape_arr = jnp.asarray(output_shape).astype(jnp.int32)
    zo = jnp.sum(shape_arr) - jnp.int32(B + HO + WO + C)
    zo16 = jnp.broadcast_to(zo, (16,)).astype(jnp.int32)
    return _unpool(inputs, pooling_indices.astype(jnp.int32), zo16)


# speedup vs baseline: 1.3782x; 1.3506x over previous
import jax
import jax.numpy as jnp
from jax import lax
from jax.experimental import pallas as pl
from jax.experimental.pallas import tpu as pltpu
from jax.experimental.pallas import tpu_sc as plsc

B, H, W, C = 8, 112, 112, 96
HO, WO = 224, 224
CW = 16
NCG = C // CW
NSUB = 16
HPT = H // NSUB
YPT = HO // NSUB
POS = HPT * W
NROW = POS // 8
CHUNK = HO * WO * CW
SLICE = CHUNK // NSUB
NZ = SLICE // 16
ROWW = WO * CW


NCHUNK = B // 2 * NCG


def _body(x_hbm, idx_hbm, zo_hbm, out_hbm,
          raw_val, raw_idx, svals, soffs, zeros, zo_v, chunk,
          sem_in, sem_sc, sem_fl):
    c = lax.axis_index("c")
    s = lax.axis_index("s")
    lane = lax.iota(jnp.int32, 16)
    h0 = s * HPT
    y0 = s * YPT

    @pl.loop(0, NZ // 16)
    def _zero_init(i):
        zeros[pl.ds(i * 16, 16)] = jnp.zeros((16,), jnp.float32)

    pltpu.sync_copy(zo_hbm, zo_v)
    zo = zo_v[...]

    def fire_loads(k, p):
        b = c * (B // 2) + k // NCG
        cg = k - (k // NCG) * NCG
        pltpu.async_copy(
            x_hbm.at[cg, b, pl.ds(h0, HPT), :], raw_val.at[p], sem_in)
        pltpu.async_copy(
            idx_hbm.at[cg, b, pl.ds(h0, HPT), :], raw_idx.at[p], sem_in)

    def wait_loads(p):
        pltpu.make_async_copy(
            x_hbm.at[0, 0, pl.ds(0, HPT), :], raw_val.at[p], sem_in).wait()
        pltpu.make_async_copy(
            idx_hbm.at[0, 0, pl.ds(0, HPT), :], raw_idx.at[p], sem_in).wait()

    fire_loads(0, 0)
    for z in range(16):
        pltpu.sync_copy(zeros, chunk.at[pl.ds(s * SLICE + z * NZ, NZ)])
    plsc.subcore_barrier()

    @pl.loop(0, NCHUNK)
    def _chunk_loop(k):
        b = c * (B // 2) + k // NCG
        cg = k - (k // NCG) * NCG
        p = k - (k // 2) * 2

        wait_loads(p)

        @pl.when(k < NCHUNK - 1)
        def _prefetch():
            fire_loads(k + 1, 1 - p)

        c5 = jnp.full((16,), 5, jnp.int32)
        c9 = jnp.full((16,), 9, jnp.int32)
        c511 = jnp.full((16,), 511, jnp.int32)
        c170 = jnp.full((16,), 170, jnp.int32)
        cmag = jnp.full((16,), 10923, jnp.int32)
        c15 = jnp.full((16,), 15, jnp.int32)
        c16 = jnp.full((16,), CW, jnp.int32)

        @pl.loop(0, HPT)
        def _h_loop(hh):
            @pl.loop(0, W // 8)
            def _w_loop(wb):
                row = hh * (W // 8) + wb
                for pp in range(8):
                    vi = raw_idx[p, hh, pl.ds((wb * 8 + pp) * CW, 16)]
                    vv = raw_val[p, hh, pl.ds((wb * 8 + pp) * CW, 16)]
                    a = lax.shift_right_logical(lax.add(vi, zo), c5)
                    b2 = lax.shift_right_logical(a, c9)
                    t = lax.add(lax.add(b2, b2), lax.bitwise_and(a, c511))
                    q = lax.add(
                        lax.mul(b2, c170),
                        lax.shift_right_logical(lax.mul(t, cmag), c15))
                    off = lax.add(lax.mul(q, c16), lane)
                    soffs[row, pl.ds(pp * 16, 16)] = off
                    svals[row, pl.ds(pp * 16, 16)] = vv
                pltpu.async_copy(svals.at[row], chunk.at[soffs.at[row]],
                                 sem_sc, add=True)

        @pl.loop(0, NROW)
        def _drain(r):
            pltpu.make_async_copy(svals.at[0], chunk.at[soffs.at[0]],
                                  sem_sc).wait()

        plsc.subcore_barrier()

        @pl.loop(0, YPT)
        def _flush_fire(yy):
            pltpu.async_copy(chunk.at[pl.ds((y0 + yy) * ROWW, ROWW)],
                             out_hbm.at[cg, b, y0 + yy, :], sem_fl)

        @pl.loop(0, YPT)
        def _flush_drain(yy):
            pltpu.make_async_copy(chunk.at[pl.ds(y0 * ROWW, ROWW)],
                                  out_hbm.at[cg, b, y0, :], sem_fl).wait()

        @pl.loop(0, 16)
        def _zero_fire(z):
            pltpu.async_copy(zeros, chunk.at[pl.ds(s * SLICE + z * NZ, NZ)],
                             sem_fl)

        @pl.loop(0, 16)
        def _zero_drain(z):
            pltpu.make_async_copy(zeros, chunk.at[pl.ds(s * SLICE, NZ)],
                                  sem_fl).wait()

        plsc.subcore_barrier()


@jax.jit
def _unpool(x, idx, zo16):
    x = (x.reshape(B, H, W, NCG, CW).transpose(3, 0, 1, 2, 4)
         .reshape(NCG, B, H, W * CW))
    idx = (idx.reshape(B, H, W, NCG, CW).transpose(3, 0, 1, 2, 4)
           .reshape(NCG, B, H, W * CW))
    fn = pl.kernel(
        _body,
        out_type=jax.ShapeDtypeStruct((NCG, B, HO, ROWW), jnp.float32),
        mesh=plsc.VectorSubcoreMesh(core_axis_name="c", subcore_axis_name="s"),
        compiler_params=pltpu.CompilerParams(use_tc_tiling_on_sc=False),
        scratch_types=[
            pltpu.VMEM((2, HPT, W * CW), jnp.float32),
            pltpu.VMEM((2, HPT, W * CW), jnp.int32),
            pltpu.VMEM((NROW, 128), jnp.float32),
            pltpu.VMEM((NROW, 128), jnp.int32),
            pltpu.VMEM((NZ,), jnp.float32),
            pltpu.VMEM((16,), jnp.int32),
            pltpu.VMEM_SHARED((CHUNK,), jnp.float32),
            pltpu.SemaphoreType.DMA,
            pltpu.SemaphoreType.DMA,
            pltpu.SemaphoreType.DMA,
        ],
    )
    out_t = fn(x, idx, zo16)
    return (out_t.reshape(NCG, B, HO, WO, CW)
            .transpose(1, 2, 3, 0, 4)
            .reshape(B, HO, WO, C))


def kernel(inputs, pooling_indices, output_shape):
    shape_arr = jnp.asarray(output_shape).astype(jnp.int32)
    zo = jnp.sum(shape_arr) - jnp.int32(B + HO + WO + C)
    zo16 = jnp.broadcast_to(zo, (16,)).astype(jnp.int32)
    return _unpool(inputs, pooling_indices.astype(jnp.int32), zo16)
